# baseline (speedup 1.0000x reference)
import jax
import jax.numpy as jnp
from jax.experimental import pallas as pl
from jax.experimental.pallas import tpu as pltpu


def _down_body(x_ref, w_ref, b_ref, a_ref, m_ref, o32_ref, o16_ref):
    Cin, D, H, W = x_ref.shape[1:]
    D2, H2 = D // 2, H // 2
    WB = o16_ref.shape[-2]
    L = m_ref.shape[-1]
    TDo = D2 if D2 <= 8 else (D2 // 4 if D2 % 4 == 0 else D2)
    for ck in range(D2 // TDo):
        xb = x_ref[0, :, 2 * TDo * ck:2 * TDo * (ck + 1)].astype(jnp.bfloat16)
        xt = jnp.transpose(xb, (1, 2, 3, 0))
        sl = [xt[:, :, min(max(8 * (wb - 1), 0), W - 8):, :][:, :, :8, :]
              for wb in range(WB)]
        ps = jnp.stack(sl, axis=0)
        ps = ps.reshape(WB, TDo, 2, H2, 2, 8, Cin)
        p = jnp.transpose(ps, (1, 3, 0, 2, 4, 5, 6))
        p = p.reshape(TDo * H2 * WB, 32 * Cin)
        y = jnp.dot(p, w_ref[...], preferred_element_type=jnp.float32)
        y = y + b_ref[...]
        y = jnp.where(y > 0.0, y, a_ref[...] * y)
        y4 = y.reshape(TDo, H2, WB, L) * m_ref[...]
        zh = jnp.zeros((TDo, 2, WB, L), jnp.float32)
        yf = jnp.concatenate([zh, y4, zh], axis=1)
        o32_ref[TDo * ck:TDo * (ck + 1)] = yf
        o16_ref[TDo * ck:TDo * (ck + 1)] = yf.astype(jnp.bfloat16)


def _down_conv(x_ncdhw, w_down, b_down, a_prelu):
    N, Cin, D, H, W = x_ncdhw.shape
    Co = w_down.shape[0]
    D2, H2, W2 = D // 2, H // 2, W // 2

    wt = jnp.transpose(w_down, (2, 3, 4, 1, 0))
    wd6 = jnp.zeros((2, 2, 8, Cin, 4, Co), wt.dtype)
    for ws in range(4):
        wd6 = wd6.at[:, :, 2 * ws:2 * ws + 2, :, ws, :].set(wt)
    w_band = wd6.reshape(32 * Cin, 4 * Co).astype(jnp.bfloat16)
    b128 = jnp.tile(b_down, 4).reshape(1, 4 * Co)
    a128 = jnp.tile(a_prelu, 4).reshape(1, 4 * Co)

    WB = (W2 + 8) // 4
    wpos = jnp.arange(WB)[:, None] * 4 + jnp.arange(4 * Co)[None, :] // Co - 4
    mask = ((wpos >= 0) & (wpos < W2)).astype(jnp.float32)

    rows = H2 * W2
    cost = pl.CostEstimate(
        flops=2 * N * D2 * H2 * WB * 32 * Cin * 4 * Co,
        transcendentals=0,
        bytes_accessed=4 * N * Cin * D * H * W + 8 * N * D2 * rows * Co)

    DP = 4 if D2 % 4 == 0 else 1
    pf = (H2 + 4, WB, 4 * Co)
    y32, y16 = pl.pallas_call(
        _down_body,
        out_shape=(jax.ShapeDtypeStruct((N, D2) + pf, jnp.float32),
                   jax.ShapeDtypeStruct((N, D2) + pf, jnp.bfloat16)),
        grid=(N, D2 // DP),
        in_specs=[
            pl.BlockSpec((1, Cin, 2 * DP, H, W), lambda n, d: (n, 0, d, 0, 0)),
            pl.BlockSpec((32 * Cin, 4 * Co), lambda n, d: (0, 0)),
            pl.BlockSpec((1, 4 * Co), lambda n, d: (0, 0)),
            pl.BlockSpec((1, 4 * Co), lambda n, d: (0, 0)),
            pl.BlockSpec((WB, 4 * Co), lambda n, d: (0, 0)),
        ],
        out_specs=(pl.BlockSpec((None, DP) + pf,
                                lambda n, d: (n, d, 0, 0, 0)),
                   pl.BlockSpec((None, DP) + pf,
                                lambda n, d: (n, d, 0, 0, 0))),
        compiler_params=pltpu.CompilerParams(
            dimension_semantics=("parallel", "parallel")),
        cost_estimate=cost,
    )(x_ncdhw, w_band, b128, a128, mask)
    return y32, y16


def _prep_conv_w(w_oidhw):
    wt = jnp.transpose(w_oidhw, (3, 4, 1, 2, 0))
    C = wt.shape[-1]
    w6 = jnp.zeros((5, 12, C, 5, 4, C), wt.dtype)
    for ws in range(4):
        w6 = w6.at[:, ws + 2:ws + 7, :, :, ws, :].set(wt)
    return w6.reshape(5, 12 * C, 5 * 4 * C).astype(jnp.bfloat16)


def _make_conv_body(D2, H2, W2, WB, last):
    def _body(*refs):
        if last:
            x0, w_ref, b_ref, a_ref, m_ref, res_ref, o_ref = refs
        else:
            x0, w_ref, b_ref, a_ref, m_ref, o_ref = refs
        C = m_ref.shape[-1] // 4
        xv = x0[...]
        zD = jnp.zeros((2,) + xv.shape[1:], xv.dtype)
        slab = jnp.concatenate([zD, xv, zD], axis=0)

        z = jnp.zeros_like(slab[:, :, :1])
        left = jnp.concatenate([z, slab[:, :, :-1]], axis=2)
        right = jnp.concatenate([slab[:, :, 1:], z], axis=2)
        pw = jnp.concatenate([left, slab, right], axis=-1)

        TDc = D2 // 2 if D2 % 2 == 0 else D2
        for ck in range(D2 // TDc):
            pwc = pw[TDc * ck:TDc * ck + TDc + 4]
            acc = None
            for kh in range(5):
                p = pwc[:, kh:kh + H2].reshape((TDc + 4) * H2 * WB, 12 * C)
                y = jnp.dot(p, w_ref[kh], preferred_element_type=jnp.float32)
                acc = y if acc is None else acc + y
            y4 = acc.reshape(TDc + 4, H2, WB, 20 * C)

            out = None
            for kd in range(5):
                t = y4[kd:kd + TDc, :, :, 4 * C * kd:4 * C * (kd + 1)]
                out = t if out is None else out + t
            out = out + b_ref[...]
            out = jnp.where(out > 0.0, out, a_ref[...] * out)
            out = out * m_ref[...]

            zh = jnp.zeros((TDc, 2, WB, 4 * C), out.dtype)
            out = jnp.concatenate([zh, out, zh], axis=1)
            sel = slice(TDc * ck, TDc * (ck + 1))
            if last:
                o_ref[sel] = out + res_ref[sel]
            else:
                o_ref[sel] = out.astype(jnp.bfloat16)
    return _body


def _conv5_layer(x_pf, w_oidhw, b, a_prelu, residual=None):
    N, D2, Hp, WB = x_pf.shape[:4]
    H2 = Hp - 4
    C = w_oidhw.shape[0]
    W2 = WB * 4 - 8
    last = residual is not None

    w_prep = _prep_conv_w(w_oidhw)
    b128 = jnp.tile(b, 4).reshape(1, 4 * C)
    a128 = jnp.tile(a_prelu, 4).reshape(1, 4 * C)
    wpos = jnp.arange(WB)[:, None] * 4 + jnp.arange(4 * C)[None, :] // C - 4
    mask = ((wpos >= 0) & (wpos < W2)).astype(jnp.float32)

    vol = pl.BlockSpec((None, D2, Hp, WB, 4 * C), lambda n: (n, 0, 0, 0, 0))
    in_specs = [
        vol,
        pl.BlockSpec(w_prep.shape, lambda n: (0, 0, 0)),
        pl.BlockSpec((1, 4 * C), lambda n: (0, 0)),
        pl.BlockSpec((1, 4 * C), lambda n: (0, 0)),
        pl.BlockSpec((WB, 4 * C), lambda n: (0, 0)),
    ]
    inputs = [x_pf, w_prep, b128, a128, mask]
    if last:
        in_specs.append(vol)
        inputs.append(residual)
        out_shape = jax.ShapeDtypeStruct((N, D2, Hp, WB, 4 * C), jnp.float32)
    else:
        out_shape = jax.ShapeDtypeStruct((N, D2, Hp, WB, 4 * C), jnp.bfloat16)

    cost = pl.CostEstimate(
        flops=2 * N * D2 * H2 * W2 * 125 * C * C,
        transcendentals=0,
        bytes_accessed=x_pf.size * 2 + 2 * w_prep.size
        + (8 if last else 2) * N * D2 * H2 * W2 * C)

    return pl.pallas_call(
        _make_conv_body(D2, H2, W2, WB, last),
        out_shape=out_shape,
        grid=(N,),
        in_specs=in_specs,
        out_specs=vol,
        compiler_params=pltpu.CompilerParams(
            dimension_semantics=("parallel",)),
        cost_estimate=cost,
    )(*inputs)


def _conv_on_slab(xv, w_ref, b_ref, a_ref, m_ref, D2, H2, WB, C, res_ref):
    zD = jnp.zeros((2,) + xv.shape[1:], xv.dtype)
    slab = jnp.concatenate([zD, xv, zD], axis=0)

    TDc = D2 // 3 if D2 % 3 == 0 else D2
    pieces = []
    for ck in range(D2 // TDc):
        slabc = slab[TDc * ck:TDc * ck + TDc + 4]
        z = jnp.zeros_like(slabc[:, :, :1])
        leftc = jnp.concatenate([z, slabc[:, :, :-1]], axis=2)
        rightc = jnp.concatenate([slabc[:, :, 1:], z], axis=2)
        pwc = jnp.concatenate([leftc, slabc, rightc], axis=-1)
        acc = None
        for kh in range(5):
            p = pwc[:, kh:kh + H2].reshape((TDc + 4) * H2 * WB, 12 * C)
            y = jnp.dot(p, w_ref[kh], preferred_element_type=jnp.float32)
            acc = y if acc is None else acc + y
        y4 = acc.reshape(TDc + 4, H2, WB, 20 * C)
        out = None
        for kd in range(5):
            t = y4[kd:kd + TDc, :, :, 4 * C * kd:4 * C * (kd + 1)]
            out = t if out is None else out + t
        out = out + b_ref[...]
        out = jnp.where(out > 0.0, out, a_ref[...] * out)
        out = out * m_ref[...]
        zh = jnp.zeros((TDc, 2, WB, 4 * C), out.dtype)
        out = jnp.concatenate([zh, out, zh], axis=1)
        sel = slice(TDc * ck, TDc * (ck + 1))
        if res_ref is not None:
            out = out + res_ref[sel]
        pieces.append((sel, out))
    return pieces


def _make_mega_body(Cin, D2, H2, W2, WB, C):
    L = 4 * C

    def _body(x_ref, wd_ref, bd_ref, ad_ref, w0_ref, b0_ref, a0_ref,
              w1_ref, b1_ref, a1_ref, m_ref, o_ref, pf16, res32):
        H, W = 2 * H2, 2 * W2
        TDo = 4 if D2 % 4 == 0 else D2

        def dchunk(ck, carry):
            xb = x_ref[0, :, pl.ds(2 * TDo * ck, 2 * TDo)]
            xt = jnp.transpose(xb, (1, 2, 3, 0))
            sl = [xt[:, :, min(max(8 * (wb - 1), 0), W - 8):, :][:, :, :8, :]
                  for wb in range(WB)]
            ps = jnp.stack(sl, axis=0)
            ps = ps.reshape(WB, TDo, 2, H2, 2, 8, Cin)
            p = jnp.transpose(ps, (1, 3, 0, 2, 4, 5, 6))
            p = p.reshape(TDo * H2 * WB, 32 * Cin)
            y = jnp.dot(p, wd_ref[...], preferred_element_type=jnp.float32)
            y = y + bd_ref[...]
            y = jnp.where(y > 0.0, y, ad_ref[...] * y)
            y4 = y.reshape(TDo, H2, WB, L) * m_ref[...]
            zh = jnp.zeros((TDo, 2, WB, L), jnp.float32)
            yf = jnp.concatenate([zh, y4, zh], axis=1)
            res32[pl.ds(TDo * ck, TDo)] = yf
            pf16[pl.ds(TDo * ck, TDo)] = yf.astype(jnp.bfloat16)
            return carry

        jax.lax.fori_loop(0, D2 // TDo, dchunk, 0)

        xv0 = pf16[...]
        for sel, v in _conv_on_slab(xv0, w0_ref, b0_ref, a0_ref, m_ref,
                                    D2, H2, WB, C, None):
            pf16[sel] = v.astype(jnp.bfloat16)
        for sel, v in _conv_on_slab(pf16[...], w1_ref, b1_ref, a1_ref, m_ref,
                                    D2, H2, WB, C, res32):
            o_ref[sel] = v
    return _body


def kernel(x, down_w, down_b, prelu1, conv_w0, conv_b0, conv_a0,
           conv_w1, conv_b1, conv_a1):
    N, Cin, D, H, W = x.shape
    Co = down_w.shape[0]
    C = Co
    D2, H2, W2 = D // 2, H // 2, W // 2
    WB = (W2 + 8) // 4
    L = 4 * C

    wt = jnp.transpose(down_w, (2, 3, 4, 1, 0))
    wd6 = jnp.zeros((2, 2, 8, Cin, 4, Co), wt.dtype)
    for ws in range(4):
        wd6 = wd6.at[:, :, 2 * ws:2 * ws + 2, :, ws, :].set(wt)
    w_band = wd6.reshape(32 * Cin, L).astype(jnp.bfloat16)

    w0p = _prep_conv_w(conv_w0)
    w1p = _prep_conv_w(conv_w1)
    bd = jnp.tile(down_b, 4).reshape(1, L)
    ad = jnp.tile(prelu1, 4).reshape(1, L)
    b0 = jnp.tile(conv_b0, 4).reshape(1, L)
    a0 = jnp.tile(conv_a0, 4).reshape(1, L)
    b1 = jnp.tile(conv_b1, 4).reshape(1, L)
    a1 = jnp.tile(conv_a1, 4).reshape(1, L)
    wpos = jnp.arange(WB)[:, None] * 4 + jnp.arange(L)[None, :] // C - 4
    mask = ((wpos >= 0) & (wpos < W2)).astype(jnp.float32)

    rvec = lambda n: (0, 0)
    cost = pl.CostEstimate(
        flops=2 * N * D2 * H2 * W2 * C * (8 * Cin + 250 * C),
        transcendentals=0,
        bytes_accessed=4 * N * Cin * D * H * W + 8 * N * D2 * H2 * W2 * C)

    pf = (D2, H2 + 4, WB, L)
    out = pl.pallas_call(
        _make_mega_body(Cin, D2, H2, W2, WB, C),
        out_shape=jax.ShapeDtypeStruct((N,) + pf, jnp.float32),
        grid=(N,),
        in_specs=[
            pl.BlockSpec((1, Cin, D, H, W), lambda n: (n, 0, 0, 0, 0)),
            pl.BlockSpec(w_band.shape, rvec),
            pl.BlockSpec(bd.shape, rvec),
            pl.BlockSpec(ad.shape, rvec),
            pl.BlockSpec(w0p.shape, lambda n: (0, 0, 0)),
            pl.BlockSpec(b0.shape, rvec),
            pl.BlockSpec(a0.shape, rvec),
            pl.BlockSpec(w1p.shape, lambda n: (0, 0, 0)),
            pl.BlockSpec(b1.shape, rvec),
            pl.BlockSpec(a1.shape, rvec),
            pl.BlockSpec(mask.shape, rvec),
        ],
        out_specs=pl.BlockSpec((None,) + pf, lambda n: (n, 0, 0, 0, 0)),
        scratch_shapes=[
            pltpu.VMEM(pf, jnp.bfloat16),
            pltpu.VMEM(pf, jnp.float32),
        ],
        compiler_params=pltpu.CompilerParams(
            dimension_semantics=("parallel",)),
        cost_estimate=cost,
    )(x.astype(jnp.bfloat16), w_band, bd, ad, w0p, b0, a0, w1p, b1, a1, mask)

    out = out.reshape(N, D2, H2 + 4, 4 * WB, C)[:, :, 2:2 + H2, 4:4 + W2, :]
    return jnp.transpose(out, (0, 4, 1, 2, 3))


# speedup vs baseline: 1.0159x; 1.0159x over previous
import jax
import jax.numpy as jnp
from jax.experimental import pallas as pl
from jax.experimental.pallas import tpu as pltpu


def _down_body(x_ref, w_ref, b_ref, a_ref, m_ref, o32_ref, o16_ref):
    Cin, D, H, W = x_ref.shape[1:]
    D2, H2 = D // 2, H // 2
    WB = o16_ref.shape[-2]
    L = m_ref.shape[-1]
    TDo = D2 if D2 <= 8 else (D2 // 4 if D2 % 4 == 0 else D2)
    for ck in range(D2 // TDo):
        xb = x_ref[0, :, 2 * TDo * ck:2 * TDo * (ck + 1)].astype(jnp.bfloat16)
        xt = jnp.transpose(xb, (1, 2, 3, 0))
        sl = [xt[:, :, min(max(8 * (wb - 1), 0), W - 8):, :][:, :, :8, :]
              for wb in range(WB)]
        ps = jnp.stack(sl, axis=0)
        ps = ps.reshape(WB, TDo, 2, H2, 2, 8, Cin)
        p = jnp.transpose(ps, (1, 3, 0, 2, 4, 5, 6))
        p = p.reshape(TDo * H2 * WB, 32 * Cin)
        y = jnp.dot(p, w_ref[...], preferred_element_type=jnp.float32)
        y = y + b_ref[...]
        y = jnp.where(y > 0.0, y, a_ref[...] * y)
        y4 = y.reshape(TDo, H2, WB, L) * m_ref[...]
        zh = jnp.zeros((TDo, 2, WB, L), jnp.float32)
        yf = jnp.concatenate([zh, y4, zh], axis=1)
        o32_ref[TDo * ck:TDo * (ck + 1)] = yf
        o16_ref[TDo * ck:TDo * (ck + 1)] = yf.astype(jnp.bfloat16)


def _down_conv(x_ncdhw, w_down, b_down, a_prelu):
    N, Cin, D, H, W = x_ncdhw.shape
    Co = w_down.shape[0]
    D2, H2, W2 = D // 2, H // 2, W // 2

    wt = jnp.transpose(w_down, (2, 3, 4, 1, 0))
    wd6 = jnp.zeros((2, 2, 8, Cin, 4, Co), wt.dtype)
    for ws in range(4):
        wd6 = wd6.at[:, :, 2 * ws:2 * ws + 2, :, ws, :].set(wt)
    w_band = wd6.reshape(32 * Cin, 4 * Co).astype(jnp.bfloat16)
    b128 = jnp.tile(b_down, 4).reshape(1, 4 * Co)
    a128 = jnp.tile(a_prelu, 4).reshape(1, 4 * Co)

    WB = (W2 + 8) // 4
    wpos = jnp.arange(WB)[:, None] * 4 + jnp.arange(4 * Co)[None, :] // Co - 4
    mask = ((wpos >= 0) & (wpos < W2)).astype(jnp.float32)

    rows = H2 * W2
    cost = pl.CostEstimate(
        flops=2 * N * D2 * H2 * WB * 32 * Cin * 4 * Co,
        transcendentals=0,
        bytes_accessed=4 * N * Cin * D * H * W + 8 * N * D2 * rows * Co)

    DP = 4 if D2 % 4 == 0 else 1
    pf = (H2 + 4, WB, 4 * Co)
    y32, y16 = pl.pallas_call(
        _down_body,
        out_shape=(jax.ShapeDtypeStruct((N, D2) + pf, jnp.float32),
                   jax.ShapeDtypeStruct((N, D2) + pf, jnp.bfloat16)),
        grid=(N, D2 // DP),
        in_specs=[
            pl.BlockSpec((1, Cin, 2 * DP, H, W), lambda n, d: (n, 0, d, 0, 0)),
            pl.BlockSpec((32 * Cin, 4 * Co), lambda n, d: (0, 0)),
            pl.BlockSpec((1, 4 * Co), lambda n, d: (0, 0)),
            pl.BlockSpec((1, 4 * Co), lambda n, d: (0, 0)),
            pl.BlockSpec((WB, 4 * Co), lambda n, d: (0, 0)),
        ],
        out_specs=(pl.BlockSpec((None, DP) + pf,
                                lambda n, d: (n, d, 0, 0, 0)),
                   pl.BlockSpec((None, DP) + pf,
                                lambda n, d: (n, d, 0, 0, 0))),
        compiler_params=pltpu.CompilerParams(
            dimension_semantics=("parallel", "parallel")),
        cost_estimate=cost,
    )(x_ncdhw, w_band, b128, a128, mask)
    return y32, y16


def _prep_conv_w(w_oidhw):
    wt = jnp.transpose(w_oidhw, (3, 4, 1, 2, 0))
    C = wt.shape[-1]
    w6 = jnp.zeros((5, 12, C, 5, 4, C), wt.dtype)
    for ws in range(4):
        w6 = w6.at[:, ws + 2:ws + 7, :, :, ws, :].set(wt)
    return w6.reshape(5, 12 * C, 5 * 4 * C).astype(jnp.bfloat16)


def _make_conv_body(D2, H2, W2, WB, last):
    def _body(*refs):
        if last:
            x0, w_ref, b_ref, a_ref, m_ref, res_ref, o_ref = refs
        else:
            x0, w_ref, b_ref, a_ref, m_ref, o_ref = refs
        C = m_ref.shape[-1] // 4
        xv = x0[...]
        zD = jnp.zeros((2,) + xv.shape[1:], xv.dtype)
        slab = jnp.concatenate([zD, xv, zD], axis=0)

        z = jnp.zeros_like(slab[:, :, :1])
        left = jnp.concatenate([z, slab[:, :, :-1]], axis=2)
        right = jnp.concatenate([slab[:, :, 1:], z], axis=2)
        pw = jnp.concatenate([left, slab, right], axis=-1)

        TDc = D2 // 2 if D2 % 2 == 0 else D2
        for ck in range(D2 // TDc):
            pwc = pw[TDc * ck:TDc * ck + TDc + 4]
            acc = None
            for kh in range(5):
                p = pwc[:, kh:kh + H2].reshape((TDc + 4) * H2 * WB, 12 * C)
                y = jnp.dot(p, w_ref[kh], preferred_element_type=jnp.float32)
                acc = y if acc is None else acc + y
            y4 = acc.reshape(TDc + 4, H2, WB, 20 * C)

            out = None
            for kd in range(5):
                t = y4[kd:kd + TDc, :, :, 4 * C * kd:4 * C * (kd + 1)]
                out = t if out is None else out + t
            out = out + b_ref[...]
            out = jnp.where(out > 0.0, out, a_ref[...] * out)
            out = out * m_ref[...]

            zh = jnp.zeros((TDc, 2, WB, 4 * C), out.dtype)
            out = jnp.concatenate([zh, out, zh], axis=1)
            sel = slice(TDc * ck, TDc * (ck + 1))
            if last:
                o_ref[sel] = out + res_ref[sel]
            else:
                o_ref[sel] = out.astype(jnp.bfloat16)
    return _body


def _conv5_layer(x_pf, w_oidhw, b, a_prelu, residual=None):
    N, D2, Hp, WB = x_pf.shape[:4]
    H2 = Hp - 4
    C = w_oidhw.shape[0]
    W2 = WB * 4 - 8
    last = residual is not None

    w_prep = _prep_conv_w(w_oidhw)
    b128 = jnp.tile(b, 4).reshape(1, 4 * C)
    a128 = jnp.tile(a_prelu, 4).reshape(1, 4 * C)
    wpos = jnp.arange(WB)[:, None] * 4 + jnp.arange(4 * C)[None, :] // C - 4
    mask = ((wpos >= 0) & (wpos < W2)).astype(jnp.float32)

    vol = pl.BlockSpec((None, D2, Hp, WB, 4 * C), lambda n: (n, 0, 0, 0, 0))
    in_specs = [
        vol,
        pl.BlockSpec(w_prep.shape, lambda n: (0, 0, 0)),
        pl.BlockSpec((1, 4 * C), lambda n: (0, 0)),
        pl.BlockSpec((1, 4 * C), lambda n: (0, 0)),
        pl.BlockSpec((WB, 4 * C), lambda n: (0, 0)),
    ]
    inputs = [x_pf, w_prep, b128, a128, mask]
    if last:
        in_specs.append(vol)
        inputs.append(residual)
        out_shape = jax.ShapeDtypeStruct((N, D2, Hp, WB, 4 * C), jnp.float32)
    else:
        out_shape = jax.ShapeDtypeStruct((N, D2, Hp, WB, 4 * C), jnp.bfloat16)

    cost = pl.CostEstimate(
        flops=2 * N * D2 * H2 * W2 * 125 * C * C,
        transcendentals=0,
        bytes_accessed=x_pf.size * 2 + 2 * w_prep.size
        + (8 if last else 2) * N * D2 * H2 * W2 * C)

    return pl.pallas_call(
        _make_conv_body(D2, H2, W2, WB, last),
        out_shape=out_shape,
        grid=(N,),
        in_specs=in_specs,
        out_specs=vol,
        compiler_params=pltpu.CompilerParams(
            dimension_semantics=("parallel",)),
        cost_estimate=cost,
    )(*inputs)


def _conv_on_slab(xv, w_ref, b_ref, a_ref, m_ref, D2, H2, WB, C, res_ref):
    zD = jnp.zeros((2,) + xv.shape[1:], xv.dtype)
    slab = jnp.concatenate([zD, xv, zD], axis=0)

    TDc = D2 // 3 if D2 % 3 == 0 else D2
    pieces = []
    for ck in range(D2 // TDc):
        slabc = slab[TDc * ck:TDc * ck + TDc + 4]
        z = jnp.zeros_like(slabc[:, :, :1])
        leftc = jnp.concatenate([z, slabc[:, :, :-1]], axis=2)
        rightc = jnp.concatenate([slabc[:, :, 1:], z], axis=2)
        pwc = jnp.concatenate([leftc, slabc, rightc], axis=-1)
        acc = None
        for kh in range(5):
            p = pwc[:, kh:kh + H2].reshape((TDc + 4) * H2 * WB, 12 * C)
            y = jnp.dot(p, w_ref[kh], preferred_element_type=jnp.float32)
            acc = y if acc is None else acc + y
        y4 = acc.reshape(TDc + 4, H2, WB, 20 * C)
        out = None
        for kd in range(5):
            t = y4[kd:kd + TDc, :, :, 4 * C * kd:4 * C * (kd + 1)]
            out = t if out is None else out + t
        out = out + b_ref[...]
        out = jnp.where(out > 0.0, out, a_ref[...] * out)
        out = out * m_ref[...]
        zh = jnp.zeros((TDc, 2, WB, 4 * C), out.dtype)
        out = jnp.concatenate([zh, out, zh], axis=1)
        sel = slice(TDc * ck, TDc * (ck + 1))
        if res_ref is not None:
            out = out + res_ref[sel]
        pieces.append((sel, out))
    return pieces


def _make_convs_body(D2, H2, W2, WB, C):
    def _body(x0, w0_ref, b0_ref, a0_ref, w1_ref, b1_ref, a1_ref, m_ref,
              res_ref, o_ref, h0s):
        xv = x0[...]
        for sel, v in _conv_on_slab(xv, w0_ref, b0_ref, a0_ref, m_ref,
                                    D2, H2, WB, C, None):
            h0s[sel] = v.astype(jnp.bfloat16)
        for sel, v in _conv_on_slab(h0s[...], w1_ref, b1_ref, a1_ref, m_ref,
                                    D2, H2, WB, C, res_ref):
            o_ref[sel] = v
    return _body


def kernel(x, down_w, down_b, prelu1, conv_w0, conv_b0, conv_a0,
           conv_w1, conv_b1, conv_a1):
    N, Cin, D, H, W = x.shape
    C = down_w.shape[0]
    D2, H2, W2 = D // 2, H // 2, W // 2
    WB = (W2 + 8) // 4
    L = 4 * C

    res32, down16 = _down_conv(x, down_w, down_b, prelu1)

    w0p = _prep_conv_w(conv_w0)
    w1p = _prep_conv_w(conv_w1)
    b0 = jnp.tile(conv_b0, 4).reshape(1, L)
    a0 = jnp.tile(conv_a0, 4).reshape(1, L)
    b1 = jnp.tile(conv_b1, 4).reshape(1, L)
    a1 = jnp.tile(conv_a1, 4).reshape(1, L)
    wpos = jnp.arange(WB)[:, None] * 4 + jnp.arange(L)[None, :] // C - 4
    mask = ((wpos >= 0) & (wpos < W2)).astype(jnp.float32)

    pf = (D2, H2 + 4, WB, L)
    vol = pl.BlockSpec((None,) + pf, lambda n: (n, 0, 0, 0, 0))
    rvec = lambda n: (0, 0)
    cost = pl.CostEstimate(
        flops=2 * N * D2 * H2 * W2 * 250 * C * C,
        transcendentals=0,
        bytes_accessed=3 * N * D2 * (H2 + 4) * WB * L * 4)

    out = pl.pallas_call(
        _make_convs_body(D2, H2, W2, WB, C),
        out_shape=jax.ShapeDtypeStruct((N,) + pf, jnp.float32),
        grid=(N,),
        in_specs=[
            vol,
            pl.BlockSpec(w0p.shape, lambda n: (0, 0, 0)),
            pl.BlockSpec(b0.shape, rvec),
            pl.BlockSpec(a0.shape, rvec),
            pl.BlockSpec(w1p.shape, lambda n: (0, 0, 0)),
            pl.BlockSpec(b1.shape, rvec),
            pl.BlockSpec(a1.shape, rvec),
            pl.BlockSpec(mask.shape, rvec),
            vol,
        ],
        out_specs=vol,
        scratch_shapes=[pltpu.VMEM(pf, jnp.bfloat16)],
        compiler_params=pltpu.CompilerParams(
            dimension_semantics=("parallel",)),
        cost_estimate=cost,
    )(down16, w0p, b0, a0, w1p, b1, a1, mask, res32)

    out = out.reshape(N, D2, H2 + 4, 4 * WB, C)[:, :, 2:2 + H2, 4:4 + W2, :]
    return jnp.transpose(out, (0, 4, 1, 2, 3))


# speedup vs baseline: 1.1532x; 1.1352x over previous
import jax
import jax.numpy as jnp
from jax.experimental import pallas as pl
from jax.experimental.pallas import tpu as pltpu


def _down_body(x_ref, w_ref, b_ref, a_ref, m_ref, o32_ref, o16_ref):
    Cin, D, H, W = x_ref.shape[1:]
    D2, H2 = D // 2, H // 2
    WB = o16_ref.shape[-2]
    L = m_ref.shape[-1]
    TDo = D2 if D2 <= 8 else (D2 // 4 if D2 % 4 == 0 else D2)
    for ck in range(D2 // TDo):
        xb = x_ref[0, :, 2 * TDo * ck:2 * TDo * (ck + 1)].astype(jnp.bfloat16)
        xt = jnp.transpose(xb, (1, 2, 3, 0))
        sl = [xt[:, :, min(max(8 * (wb - 1), 0), W - 8):, :][:, :, :8, :]
              for wb in range(WB)]
        ps = jnp.stack(sl, axis=0)
        ps = ps.reshape(WB, TDo, 2, H2, 2, 8, Cin)
        p = jnp.transpose(ps, (1, 3, 0, 2, 4, 5, 6))
        p = p.reshape(TDo * H2 * WB, 32 * Cin)
        y = jnp.dot(p, w_ref[...], preferred_element_type=jnp.float32)
        y = y + b_ref[...]
        y = jnp.where(y > 0.0, y, a_ref[...] * y)
        y4 = y.reshape(TDo, H2, WB, L) * m_ref[...]
        zh = jnp.zeros((TDo, 2, WB, L), jnp.float32)
        yf = jnp.concatenate([zh, y4, zh], axis=1)
        o32_ref[TDo * ck:TDo * (ck + 1)] = yf
        o16_ref[TDo * ck:TDo * (ck + 1)] = yf.astype(jnp.bfloat16)


def _down_conv(x_ncdhw, w_down, b_down, a_prelu):
    N, Cin, D, H, W = x_ncdhw.shape
    Co = w_down.shape[0]
    D2, H2, W2 = D // 2, H // 2, W // 2

    wt = jnp.transpose(w_down, (2, 3, 4, 1, 0))
    wd6 = jnp.zeros((2, 2, 8, Cin, 4, Co), wt.dtype)
    for ws in range(4):
        wd6 = wd6.at[:, :, 2 * ws:2 * ws + 2, :, ws, :].set(wt)
    w_band = wd6.reshape(32 * Cin, 4 * Co).astype(jnp.bfloat16)
    b128 = jnp.tile(b_down, 4).reshape(1, 4 * Co)
    a128 = jnp.tile(a_prelu, 4).reshape(1, 4 * Co)

    WB = (W2 + 8) // 4
    wpos = jnp.arange(WB)[:, None] * 4 + jnp.arange(4 * Co)[None, :] // Co - 4
    mask = ((wpos >= 0) & (wpos < W2)).astype(jnp.float32)

    rows = H2 * W2
    cost = pl.CostEstimate(
        flops=2 * N * D2 * H2 * WB * 32 * Cin * 4 * Co,
        transcendentals=0,
        bytes_accessed=4 * N * Cin * D * H * W + 8 * N * D2 * rows * Co)

    DP = 4 if D2 % 4 == 0 else 1
    pf = (H2 + 4, WB, 4 * Co)
    y32, y16 = pl.pallas_call(
        _down_body,
        out_shape=(jax.ShapeDtypeStruct((N, D2) + pf, jnp.float32),
                   jax.ShapeDtypeStruct((N, D2) + pf, jnp.bfloat16)),
        grid=(N, D2 // DP),
        in_specs=[
            pl.BlockSpec((1, Cin, 2 * DP, H, W), lambda n, d: (n, 0, d, 0, 0)),
            pl.BlockSpec((32 * Cin, 4 * Co), lambda n, d: (0, 0)),
            pl.BlockSpec((1, 4 * Co), lambda n, d: (0, 0)),
            pl.BlockSpec((1, 4 * Co), lambda n, d: (0, 0)),
            pl.BlockSpec((WB, 4 * Co), lambda n, d: (0, 0)),
        ],
        out_specs=(pl.BlockSpec((None, DP) + pf,
                                lambda n, d: (n, d, 0, 0, 0)),
                   pl.BlockSpec((None, DP) + pf,
                                lambda n, d: (n, d, 0, 0, 0))),
        compiler_params=pltpu.CompilerParams(
            dimension_semantics=("parallel", "parallel")),
        cost_estimate=cost,
    )(x_ncdhw, w_band, b128, a128, mask)
    return y32, y16


def _prep_conv_w(w_oidhw):
    wt = jnp.transpose(w_oidhw, (3, 4, 1, 2, 0))
    C = wt.shape[-1]
    w6 = jnp.zeros((5, 12, C, 5, 4, C), wt.dtype)
    for ws in range(4):
        w6 = w6.at[:, ws + 2:ws + 7, :, :, ws, :].set(wt)
    return w6.reshape(5, 12 * C, 5 * 4 * C).astype(jnp.bfloat16)


def _make_conv_body(D2, H2, W2, WB, last):
    def _body(*refs):
        if last:
            x0, w_ref, b_ref, a_ref, m_ref, res_ref, o_ref = refs
        else:
            x0, w_ref, b_ref, a_ref, m_ref, o_ref = refs
        C = m_ref.shape[-1] // 4
        xv = x0[...]
        zD = jnp.zeros((2,) + xv.shape[1:], xv.dtype)
        slab = jnp.concatenate([zD, xv, zD], axis=0)

        z = jnp.zeros_like(slab[:, :, :1])
        left = jnp.concatenate([z, slab[:, :, :-1]], axis=2)
        right = jnp.concatenate([slab[:, :, 1:], z], axis=2)
        pw = jnp.concatenate([left, slab, right], axis=-1)

        TDc = D2 // 2 if D2 % 2 == 0 else D2
        for ck in range(D2 // TDc):
            pwc = pw[TDc * ck:TDc * ck + TDc + 4]
            acc = None
            for kh in range(5):
                p = pwc[:, kh:kh + H2].reshape((TDc + 4) * H2 * WB, 12 * C)
                y = jnp.dot(p, w_ref[kh], preferred_element_type=jnp.float32)
                acc = y if acc is None else acc + y
            y4 = acc.reshape(TDc + 4, H2, WB, 20 * C)

            out = None
            for kd in range(5):
                t = y4[kd:kd + TDc, :, :, 4 * C * kd:4 * C * (kd + 1)]
                out = t if out is None else out + t
            out = out + b_ref[...]
            out = jnp.where(out > 0.0, out, a_ref[...] * out)
            out = out * m_ref[...]

            zh = jnp.zeros((TDc, 2, WB, 4 * C), out.dtype)
            out = jnp.concatenate([zh, out, zh], axis=1)
            sel = slice(TDc * ck, TDc * (ck + 1))
            if last:
                o_ref[sel] = out + res_ref[sel]
            else:
                o_ref[sel] = out.astype(jnp.bfloat16)
    return _body


def _conv5_layer(x_pf, w_oidhw, b, a_prelu, residual=None):
    N, D2, Hp, WB = x_pf.shape[:4]
    H2 = Hp - 4
    C = w_oidhw.shape[0]
    W2 = WB * 4 - 8
    last = residual is not None

    w_prep = _prep_conv_w(w_oidhw)
    b128 = jnp.tile(b, 4).reshape(1, 4 * C)
    a128 = jnp.tile(a_prelu, 4).reshape(1, 4 * C)
    wpos = jnp.arange(WB)[:, None] * 4 + jnp.arange(4 * C)[None, :] // C - 4
    mask = ((wpos >= 0) & (wpos < W2)).astype(jnp.float32)

    vol = pl.BlockSpec((None, D2, Hp, WB, 4 * C), lambda n: (n, 0, 0, 0, 0))
    in_specs = [
        vol,
        pl.BlockSpec(w_prep.shape, lambda n: (0, 0, 0)),
        pl.BlockSpec((1, 4 * C), lambda n: (0, 0)),
        pl.BlockSpec((1, 4 * C), lambda n: (0, 0)),
        pl.BlockSpec((WB, 4 * C), lambda n: (0, 0)),
    ]
    inputs = [x_pf, w_prep, b128, a128, mask]
    if last:
        in_specs.append(vol)
        inputs.append(residual)
        out_shape = jax.ShapeDtypeStruct((N, D2, Hp, WB, 4 * C), jnp.float32)
    else:
        out_shape = jax.ShapeDtypeStruct((N, D2, Hp, WB, 4 * C), jnp.bfloat16)

    cost = pl.CostEstimate(
        flops=2 * N * D2 * H2 * W2 * 125 * C * C,
        transcendentals=0,
        bytes_accessed=x_pf.size * 2 + 2 * w_prep.size
        + (8 if last else 2) * N * D2 * H2 * W2 * C)

    return pl.pallas_call(
        _make_conv_body(D2, H2, W2, WB, last),
        out_shape=out_shape,
        grid=(N,),
        in_specs=in_specs,
        out_specs=vol,
        compiler_params=pltpu.CompilerParams(
            dimension_semantics=("parallel",)),
        cost_estimate=cost,
    )(*inputs)


def kernel(x, down_w, down_b, prelu1, conv_w0, conv_b0, conv_a0,
           conv_w1, conv_b1, conv_a1):
    res32, down16 = _down_conv(x, down_w, down_b, prelu1)
    N, D2 = down16.shape[:2]
    C = down_w.shape[0]
    H2, W2 = x.shape[3] // 2, x.shape[4] // 2

    h0 = _conv5_layer(down16, conv_w0, conv_b0, conv_a0)
    out = _conv5_layer(h0, conv_w1, conv_b1, conv_a1, residual=res32)
    WB = out.shape[3]
    out = out.reshape(N, D2, H2 + 4, 4 * WB, C)[:, :, 2:2 + H2, 4:4 + W2, :]
    return jnp.transpose(out, (0, 4, 1, 2, 3))
